# Initial kernel scaffold; baseline (speedup 1.0000x reference)
import jax
import jax.numpy as jnp
from jax import lax
from jax.experimental import pallas as pl
from jax.experimental.pallas import tpu as pltpu


def _gather2_kernel(idx_ref, table_ref, out_ref):
    p_rows = out_ref.shape[0]
    num_hi, wide = table_ref.shape
    d = wide // 4

    iota = lax.broadcasted_iota(jnp.int32, (p_rows, num_hi), 1)
    table = table_ref[...]
    sels = []
    for e in range(2):
        tgt = idx_ref[:, e : e + 1]
        hi = tgt >> 2
        lo = tgt & 3
        onehot = jnp.where(iota == hi, 1.0, 0.0).astype(jnp.float32)
        partial = jnp.dot(
            onehot,
            table,
            preferred_element_type=jnp.float32,
            precision=lax.Precision.HIGHEST,
        )
        a = partial[:, : 2 * d]
        b = partial[:, 2 * d :]
        sel1 = jnp.where(lo >= 2, b, a)
        rolled = pltpu.roll(sel1, d, axis=1)
        sel2 = jnp.where((lo & 1) == 1, rolled, sel1)
        sels.append(sel2)

    lane = lax.broadcasted_iota(jnp.int32, (p_rows, 2 * d), 1)
    out_ref[...] = jnp.where(lane < d, sels[0], pltpu.roll(sels[1], d, axis=1))


def _gather2_call(idx2, table4, num_blocks_per_core, p_rows):
    n_packed = idx2.shape[0]
    num_hi, wide = table4.shape
    nb = num_blocks_per_core

    return pl.pallas_call(
        _gather2_kernel,
        out_shape=jax.ShapeDtypeStruct((n_packed, wide // 2), jnp.float32),
        grid=(2, nb),
        in_specs=[
            pl.BlockSpec((p_rows, 2), lambda c, i: (c * nb + i, 0)),
            pl.BlockSpec((num_hi, wide), lambda c, i: (0, 0)),
        ],
        out_specs=pl.BlockSpec((p_rows, wide // 2), lambda c, i: (c * nb + i, 0)),
        compiler_params=pltpu.CompilerParams(
            dimension_semantics=("core_parallel", "arbitrary"),
            vmem_limit_bytes=64 * 1024 * 1024,
        ),
    )(idx2, table4)


def kernel(user_fea, embedding_location):
    n = user_fea.shape[0]
    num_location, d = embedding_location.shape
    assert num_location % 4 == 0 and d % 2 == 0

    idx = jnp.clip(user_fea[:, 0].astype(jnp.int32), 0, num_location - 1)

    p_rows = 512
    rows_per_block = 2 * p_rows
    n_pad = ((n + 2 * rows_per_block - 1) // (2 * rows_per_block)) * (2 * rows_per_block)
    if n_pad != n:
        idx = jnp.pad(idx, (0, n_pad - n))
    idx2 = idx.reshape(n_pad // 2, 2)

    table4 = embedding_location.reshape(num_location // 4, 4 * d)

    nb_total = (n_pad // 2) // p_rows
    out = _gather2_call(idx2, table4, nb_total // 2, p_rows)
    return out.reshape(n_pad, d)[:n]


# speedup vs baseline: 2.5955x; 2.5955x over previous
import jax
import jax.numpy as jnp
from jax import lax
from jax.experimental import pallas as pl
from jax.experimental.pallas import tpu as pltpu


def _gather2_kernel(idx_ref, table_ref, out_ref):
    p_rows = out_ref.shape[0]
    num_hi, wide = table_ref.shape
    d = wide // 4

    iota = lax.broadcasted_iota(jnp.int32, (p_rows, num_hi), 1)
    table = table_ref[...]
    sels = []
    for e in range(2):
        tgt = idx_ref[:, e : e + 1]
        hi = tgt >> 2
        lo = tgt & 3
        onehot = jnp.where(iota == hi, 1.0, 0.0).astype(jnp.float32)
        partial = jnp.dot(
            onehot,
            table,
            preferred_element_type=jnp.float32,
            precision=lax.Precision.HIGHEST,
        )
        a = partial[:, : 2 * d]
        b = partial[:, 2 * d :]
        sel1 = jnp.where(lo >= 2, b, a)
        rolled = pltpu.roll(sel1, d, axis=1)
        sel2 = jnp.where((lo & 1) == 1, rolled, sel1)
        sels.append(sel2)

    lane = lax.broadcasted_iota(jnp.int32, (p_rows, 2 * d), 1)
    out_ref[...] = jnp.where(lane < d, sels[0], pltpu.roll(sels[1], d, axis=1))


def _gather2_call(idx2, table4, num_blocks, p_rows):
    n_packed = idx2.shape[0]
    num_hi, wide = table4.shape

    return pl.pallas_call(
        _gather2_kernel,
        out_shape=jax.ShapeDtypeStruct((n_packed, wide // 2), jnp.float32),
        grid=(num_blocks,),
        in_specs=[
            pl.BlockSpec((p_rows, 2), lambda i: (i, 0)),
            pl.BlockSpec((num_hi, wide), lambda i: (0, 0)),
        ],
        out_specs=pl.BlockSpec((p_rows, wide // 2), lambda i: (i, 0)),
        compiler_params=pltpu.CompilerParams(
            dimension_semantics=("arbitrary",),
            vmem_limit_bytes=64 * 1024 * 1024,
        ),
    )(idx2, table4)


def kernel(user_fea, embedding_location):
    n = user_fea.shape[0]
    num_location, d = embedding_location.shape
    assert num_location % 4 == 0 and d % 2 == 0

    idx = jnp.clip(user_fea[:, 0].astype(jnp.int32), 0, num_location - 1)

    p_rows = 512
    rows_per_block = 2 * p_rows
    n_pad = ((n + 2 * rows_per_block - 1) // (2 * rows_per_block)) * (2 * rows_per_block)
    if n_pad != n:
        idx = jnp.pad(idx, (0, n_pad - n))
    idx2 = idx.reshape(n_pad // 2, 2)

    table4 = embedding_location.reshape(num_location // 4, 4 * d)

    nb_total = (n_pad // 2) // p_rows
    out = _gather2_call(idx2, table4, nb_total, p_rows)
    return out.reshape(n_pad, d)[:n]


# speedup vs baseline: 3.8701x; 1.4911x over previous
import jax
import jax.numpy as jnp
from jax import lax
from jax.experimental import pallas as pl
from jax.experimental.pallas import tpu as pltpu

_PC = 512
_CHUNKS = 8
_P = _PC * _CHUNKS


def _gather2_kernel(idx_ref, table_ref, out_ref):
    num_hi = table_ref.shape[0]
    wide = table_ref.shape[1] // 2
    d = wide // 4

    table = table_ref[...]
    for c in range(_CHUNKS):
        iota = lax.broadcasted_iota(jnp.int32, (_PC, num_hi), 1)
        sels = []
        for e in range(2):
            tgt = idx_ref[pl.ds(c * _PC, _PC), e : e + 1]
            hi = tgt >> 2
            lo = tgt & 3
            onehot = jnp.where(iota == hi, 1.0, 0.0).astype(jnp.bfloat16)
            ab = jnp.dot(
                onehot, table, preferred_element_type=jnp.float32
            )
            partial = ab[:, :wide] + ab[:, wide:]
            a = partial[:, : 2 * d]
            b = partial[:, 2 * d :]
            sel1 = jnp.where(lo >= 2, b, a)
            rolled = pltpu.roll(sel1, d, axis=1)
            sel2 = jnp.where((lo & 1) == 1, rolled, sel1)
            sels.append(sel2)

        lane = lax.broadcasted_iota(jnp.int32, (_PC, 2 * d), 1)
        out_ref[pl.ds(c * _PC, _PC), :] = jnp.where(
            lane < d, sels[0], pltpu.roll(sels[1], d, axis=1)
        )


def _gather2_call(idx2, table_hl, num_blocks, p_rows):
    n_packed = idx2.shape[0]
    num_hi, wide2 = table_hl.shape

    return pl.pallas_call(
        _gather2_kernel,
        out_shape=jax.ShapeDtypeStruct((n_packed, wide2 // 4), jnp.float32),
        grid=(num_blocks,),
        in_specs=[
            pl.BlockSpec((p_rows, 2), lambda i: (i, 0)),
            pl.BlockSpec((num_hi, wide2), lambda i: (0, 0)),
        ],
        out_specs=pl.BlockSpec((p_rows, wide2 // 4), lambda i: (i, 0)),
        compiler_params=pltpu.CompilerParams(
            dimension_semantics=("arbitrary",),
            vmem_limit_bytes=64 * 1024 * 1024,
        ),
    )(idx2, table_hl)


def kernel(user_fea, embedding_location):
    n = user_fea.shape[0]
    num_location, d = embedding_location.shape
    assert num_location % 4 == 0 and d % 2 == 0

    idx = jnp.clip(user_fea[:, 0].astype(jnp.int32), 0, num_location - 1)

    rows_per_block = 2 * _P
    n_pad = ((n + rows_per_block - 1) // rows_per_block) * rows_per_block
    if n_pad != n:
        idx = jnp.pad(idx, (0, n_pad - n))
    idx2 = idx.reshape(n_pad // 2, 2)

    table4 = embedding_location.reshape(num_location // 4, 4 * d)
    t_hi = table4.astype(jnp.bfloat16)
    t_lo = (table4 - t_hi.astype(jnp.float32)).astype(jnp.bfloat16)
    table_hl = jnp.concatenate([t_hi, t_lo], axis=1)

    nb_total = (n_pad // 2) // _P
    out = _gather2_call(idx2, table_hl, nb_total, _P)
    return out.reshape(n_pad, d)[:n]
